# baseline (speedup 1.0000x reference)
import dataclasses
import functools

import jax
import jax.numpy as jnp
from jax import lax
from jax.experimental import pallas as pl
from jax.experimental.pallas import tpu as pltpu
from jax.experimental.pallas import tpu_sc as plsc

N = 10000
E = 320000
D_IN = 128
H = 150
HP = 160
HC = 80
L = 3
G3 = 3 * HP

NC = 2
NS = 16
EPS = E // NS
K = 80
NCHUNK = EPS // K
NBUF = 5
GLA = 4
NP = 10240
ZR = 128
RPS = NP // NS


def _mm_body(x_ref, w_ref, o_ref):
    res = jnp.dot(x_ref[...], w_ref[...], preferred_element_type=jnp.float32)
    o_ref[0] = res[:, :HC]
    o_ref[1] = res[:, HC:]


def _tc_matmul(x, w, r_blk=2000):
    return pl.pallas_call(
        _mm_body,
        grid=(N // r_blk,),
        in_specs=[pl.BlockSpec((r_blk, HP), lambda i: (i, 0)),
                  pl.BlockSpec((HP, HP), lambda i: (0, 0))],
        out_specs=pl.BlockSpec((NC, r_blk, HC), lambda i: (0, i, 0)),
        out_shape=jax.ShapeDtypeStruct((NC, N, HC), jnp.float32),
    )(x, w)


def _gru_body(p_ref, x_ref, wih_ref, whh_ref, bih_ref, bhh_ref, o_ref):
    agg = jnp.concatenate([p_ref[0], p_ref[1]], axis=-1)
    x = x_ref[...]
    gi = jnp.dot(agg, wih_ref[...],
                 preferred_element_type=jnp.float32) + bih_ref[...]
    gh = jnp.dot(x, whh_ref[...],
                 preferred_element_type=jnp.float32) + bhh_ref[...]
    r = jax.nn.sigmoid(gi[:, :HP] + gh[:, :HP])
    z = jax.nn.sigmoid(gi[:, HP:2 * HP] + gh[:, HP:2 * HP])
    n = jnp.tanh(gi[:, 2 * HP:] + r * gh[:, 2 * HP:])
    o_ref[...] = (1.0 - z) * n + z * x


def _tc_gru(p, x, wih, whh, bih, bhh, r_blk=2000):
    return pl.pallas_call(
        _gru_body,
        grid=(N // r_blk,),
        in_specs=[pl.BlockSpec((NC, r_blk, HC), lambda i: (0, i, 0)),
                  pl.BlockSpec((r_blk, HP), lambda i: (i, 0)),
                  pl.BlockSpec((HP, G3), lambda i: (0, 0)),
                  pl.BlockSpec((HP, G3), lambda i: (0, 0)),
                  pl.BlockSpec((1, G3), lambda i: (0, 0)),
                  pl.BlockSpec((1, G3), lambda i: (0, 0))],
        out_specs=pl.BlockSpec((r_blk, HP), lambda i: (i, 0)),
        out_shape=jax.ShapeDtypeStruct((N, HP), jnp.float32),
    )(p, x, wih, whh, bih, bhh)


def _edge_body(m_hbm, epk_hbm, out_hbm,
               pk0, pk1, pk2, pk3, pk4,
               rw0, rw1, rw2, rw3, rw4, zero_v, acc_sp,
               is0, is1, is2, is3, is4,
               gs0, gs1, gs2, gs3, gs4):
    pks = [pk0, pk1, pk2, pk3, pk4]
    rowss = [rw0, rw1, rw2, rw3, rw4]
    isems = [is0, is1, is2, is3, is4]
    gsems = [gs0, gs1, gs2, gs3, gs4]
    cid = lax.axis_index("c")
    sid = lax.axis_index("s")

    zvec = jnp.zeros((16,), jnp.float32)

    @pl.loop(0, ZR)
    def _(r):
        for j in range(HC // 16):
            zero_v[r, pl.ds(j * 16, 16)] = zvec

    @pl.loop(0, RPS // ZR)
    def _(t):
        pltpu.sync_copy(zero_v, acc_sp.at[pl.ds(sid * RPS + t * ZR, ZR)])

    plsc.subcore_barrier()

    cbase = sid * NCHUNK

    def idx_start(ck, pk, sem):
        pltpu.make_async_copy(epk_hbm.at[cbase + ck], pk, sem).start()

    def idx_wait(pk, sem):
        pltpu.make_async_copy(epk_hbm.at[cbase], pk, sem).wait()

    def gstart(pk, rows, sem):
        pltpu.make_async_copy(m_hbm.at[cid].at[pk.at[0]], rows, sem).start()

    def gwait(pk, rows, sem):
        pltpu.make_async_copy(m_hbm.at[cid].at[pk.at[0]], rows, sem).wait()

    def process(pk, rows):
        @plsc.parallel_loop(0, K, unroll=8)
        def _(e):
            wi = plsc.load_gather(pk, [jnp.full((16,), 2, jnp.int32),
                                       jnp.full((16,), e, jnp.int32)])
            ws = plsc.bitcast(wi, jnp.float32)
            for j in range(HC // 16):
                slc = pl.ds(j * 16, 16)
                rows[e, slc] = rows[e, slc] * ws

        pltpu.sync_copy(rows, acc_sp.at[pk.at[1]], add=True)

    for b in range(NBUF):
        idx_start(b, pks[b], isems[b])
    for b in range(GLA):
        idx_wait(pks[b], isems[b])
        gstart(pks[b], rowss[b], gsems[b])

    @pl.loop(0, NCHUNK // NBUF)
    def _(g):
        c0 = g * NBUF
        for b in range(NBUF):
            c = c0 + b
            gwait(pks[b], rowss[b], gsems[b])
            process(pks[b], rowss[b])

            @pl.when(c + NBUF < NCHUNK)
            def _():
                idx_start(c + NBUF, pks[b], isems[b])

            b3 = (b + GLA) % NBUF

            @pl.when(c + GLA < NCHUNK)
            def _():
                idx_wait(pks[b3], isems[b3])
                gstart(pks[b3], rowss[b3], gsems[b3])

    plsc.subcore_barrier()

    @pl.loop(0, RPS // ZR)
    def _(t):
        r0 = sid * RPS + t * ZR
        pltpu.sync_copy(acc_sp.at[pl.ds(r0, ZR)],
                        out_hbm.at[cid].at[pl.ds(r0, ZR)])


def _sc_compiler_params():
    cp = pltpu.CompilerParams()
    if "needs_layout_passes" in pltpu.CompilerParams.__dataclass_fields__:
        cp = dataclasses.replace(cp, needs_layout_passes=False)
    if "use_tc_tiling_on_sc" in pltpu.CompilerParams.__dataclass_fields__:
        cp = dataclasses.replace(cp, use_tc_tiling_on_sc=False)
    return cp


def _sc_edge(m, epk):
    mesh = plsc.VectorSubcoreMesh(core_axis_name="c", subcore_axis_name="s")
    f = pl.kernel(
        _edge_body,
        out_type=jax.ShapeDtypeStruct((NC, NP, HC), jnp.float32),
        mesh=mesh,
        compiler_params=_sc_compiler_params(),
        scratch_types=(
            [pltpu.VMEM((3, K), jnp.int32) for _ in range(NBUF)]
            + [pltpu.VMEM((K, HC), jnp.float32) for _ in range(NBUF)]
            + [pltpu.VMEM((ZR, HC), jnp.float32),
               pltpu.VMEM_SHARED((NP, HC), jnp.float32)]
            + [pltpu.SemaphoreType.DMA for _ in range(2 * NBUF)]
        ),
    )
    return f(m, epk)


def _pack_edges(src, dst, w):
    return jnp.stack([
        src.reshape(E // K, K),
        dst.reshape(E // K, K),
        lax.bitcast_convert_type(w, jnp.int32).reshape(E // K, K),
    ], axis=1)


def _prep_conv_w(weight):
    return jnp.pad(weight, ((0, 0), (0, HP - H), (0, HP - H)))


def _prep_gru_w(w):
    blocks = [jnp.pad(w[g * H:(g + 1) * H, :].T,
                      ((0, HP - H), (0, HP - H))) for g in range(3)]
    return jnp.concatenate(blocks, axis=1)


def _prep_gru_b(b):
    bs = [jnp.pad(b[g * H:(g + 1) * H], (0, HP - H)) for g in range(3)]
    return jnp.concatenate(bs)[None, :]


def _ggc(xp, epk, conv_w, wih, whh, bih, bhh):
    for i in range(L):
        m = _tc_matmul(xp, conv_w[i])
        p = _sc_edge(m, epk)
        xp = _tc_gru(p, xp, wih, whh, bih, bhh)
    return xp


def kernel(x, edge_index, edge_attr, enc_weight, enc_w_ih, enc_w_hh,
           enc_b_ih, enc_b_hh, dec_weight, dec_w_ih, dec_w_hh,
           dec_b_ih, dec_b_hh):
    src = edge_index[0].astype(jnp.int32)
    dst = edge_index[1].astype(jnp.int32)
    w = edge_attr.astype(jnp.float32)
    epk = _pack_edges(src, dst, w)

    xp = jnp.pad(x, ((0, 0), (0, HP - D_IN)))

    h = _ggc(xp, epk, _prep_conv_w(enc_weight),
             _prep_gru_w(enc_w_ih), _prep_gru_w(enc_w_hh),
             _prep_gru_b(enc_b_ih), _prep_gru_b(enc_b_hh))
    r = _ggc(h, epk, _prep_conv_w(dec_weight),
             _prep_gru_w(dec_w_ih), _prep_gru_w(dec_w_hh),
             _prep_gru_b(dec_b_ih), _prep_gru_b(dec_b_hh))
    return h[:, :H], r[:, :H]


# speedup vs baseline: 1.0124x; 1.0124x over previous
import dataclasses
import functools

import jax
import jax.numpy as jnp
from jax import lax
from jax.experimental import pallas as pl
from jax.experimental.pallas import tpu as pltpu
from jax.experimental.pallas import tpu_sc as plsc

N = 10000
E = 320000
D_IN = 128
H = 150
HP = 160
HC = 80
L = 3
G3 = 3 * HP

NC = 2
NS = 16
EPS = E // NS
K = 80
NCHUNK = EPS // K
NBUF = 5
GLA = 3
NP = 10240
ZR = 128
RPS = NP // NS


def _mm_body(x_ref, w_ref, o_ref):
    res = jnp.dot(x_ref[...], w_ref[...], preferred_element_type=jnp.float32)
    o_ref[0] = res[:, :HC]
    o_ref[1] = res[:, HC:]


def _tc_matmul(x, w, r_blk=2000):
    return pl.pallas_call(
        _mm_body,
        grid=(N // r_blk,),
        in_specs=[pl.BlockSpec((r_blk, HP), lambda i: (i, 0)),
                  pl.BlockSpec((HP, HP), lambda i: (0, 0))],
        out_specs=pl.BlockSpec((NC, r_blk, HC), lambda i: (0, i, 0)),
        out_shape=jax.ShapeDtypeStruct((NC, N, HC), jnp.float32),
    )(x, w)


def _gh_body(x_ref, whh_ref, bhh_ref, o_ref):
    o_ref[...] = jnp.dot(x_ref[...], whh_ref[...],
                         preferred_element_type=jnp.float32) + bhh_ref[...]


def _tc_gh(x, whh, bhh, r_blk=2000):
    return pl.pallas_call(
        _gh_body,
        grid=(N // r_blk,),
        in_specs=[pl.BlockSpec((r_blk, HP), lambda i: (i, 0)),
                  pl.BlockSpec((HP, G3), lambda i: (0, 0)),
                  pl.BlockSpec((1, G3), lambda i: (0, 0))],
        out_specs=pl.BlockSpec((r_blk, G3), lambda i: (i, 0)),
        out_shape=jax.ShapeDtypeStruct((N, G3), jnp.float32),
    )(x, whh, bhh)


def _gru_math(p_ref, x_ref, gh_ref, wih_ref, bih_ref):
    agg = jnp.concatenate([p_ref[0], p_ref[1]], axis=-1)
    x = x_ref[...]
    gi = jnp.dot(agg, wih_ref[...],
                 preferred_element_type=jnp.float32) + bih_ref[...]
    gh = gh_ref[...]
    r = jax.nn.sigmoid(gi[:, :HP] + gh[:, :HP])
    z = jax.nn.sigmoid(gi[:, HP:2 * HP] + gh[:, HP:2 * HP])
    n = jnp.tanh(gi[:, 2 * HP:] + r * gh[:, 2 * HP:])
    return (1.0 - z) * n + z * x


def _post_body(p_ref, x_ref, gh_ref, wih_ref, bih_ref, wn_ref,
               xo_ref, mo_ref):
    xn = _gru_math(p_ref, x_ref, gh_ref, wih_ref, bih_ref)
    xo_ref[...] = xn
    res = jnp.dot(xn, wn_ref[...], preferred_element_type=jnp.float32)
    mo_ref[0] = res[:, :HC]
    mo_ref[1] = res[:, HC:]


def _post_final_body(p_ref, x_ref, gh_ref, wih_ref, bih_ref, xo_ref):
    xo_ref[...] = _gru_math(p_ref, x_ref, gh_ref, wih_ref, bih_ref)


_POST_SPECS = [pl.BlockSpec((NC, 2000, HC), lambda i: (0, i, 0)),
               pl.BlockSpec((2000, HP), lambda i: (i, 0)),
               pl.BlockSpec((2000, G3), lambda i: (i, 0)),
               pl.BlockSpec((HP, G3), lambda i: (0, 0)),
               pl.BlockSpec((1, G3), lambda i: (0, 0))]


def _tc_post(p, x, gh, wih, bih, wnext):
    return pl.pallas_call(
        _post_body,
        grid=(N // 2000,),
        in_specs=_POST_SPECS + [pl.BlockSpec((HP, HP), lambda i: (0, 0))],
        out_specs=[pl.BlockSpec((2000, HP), lambda i: (i, 0)),
                   pl.BlockSpec((NC, 2000, HC), lambda i: (0, i, 0))],
        out_shape=[jax.ShapeDtypeStruct((N, HP), jnp.float32),
                   jax.ShapeDtypeStruct((NC, N, HC), jnp.float32)],
    )(p, x, gh, wih, bih, wnext)


def _tc_post_final(p, x, gh, wih, bih):
    return pl.pallas_call(
        _post_final_body,
        grid=(N // 2000,),
        in_specs=_POST_SPECS,
        out_specs=pl.BlockSpec((2000, HP), lambda i: (i, 0)),
        out_shape=jax.ShapeDtypeStruct((N, HP), jnp.float32),
    )(p, x, gh, wih, bih)


def _edge_body(m_hbm, epk_hbm, out_hbm,
               pk0, pk1, pk2, pk3, pk4,
               rw0, rw1, rw2, rw3, rw4, zero_v, acc_sp,
               is0, is1, is2, is3, is4,
               gs0, gs1, gs2, gs3, gs4):
    pks = [pk0, pk1, pk2, pk3, pk4]
    rowss = [rw0, rw1, rw2, rw3, rw4]
    isems = [is0, is1, is2, is3, is4]
    gsems = [gs0, gs1, gs2, gs3, gs4]
    cid = lax.axis_index("c")
    sid = lax.axis_index("s")

    zvec = jnp.zeros((16,), jnp.float32)

    @pl.loop(0, ZR)
    def _(r):
        for j in range(HC // 16):
            zero_v[r, pl.ds(j * 16, 16)] = zvec

    @pl.loop(0, RPS // ZR)
    def _(t):
        pltpu.sync_copy(zero_v, acc_sp.at[pl.ds(sid * RPS + t * ZR, ZR)])

    plsc.subcore_barrier()

    cbase = sid * NCHUNK

    def idx_start(ck, pk, sem):
        pltpu.make_async_copy(epk_hbm.at[cbase + ck], pk, sem).start()

    def idx_wait(pk, sem):
        pltpu.make_async_copy(epk_hbm.at[cbase], pk, sem).wait()

    def gstart(pk, rows, sem):
        pltpu.make_async_copy(m_hbm.at[cid].at[pk.at[0]], rows, sem).start()

    def gwait(pk, rows, sem):
        pltpu.make_async_copy(m_hbm.at[cid].at[pk.at[0]], rows, sem).wait()

    def process(pk, rows):
        @plsc.parallel_loop(0, K, unroll=8)
        def _(e):
            wi = plsc.load_gather(pk, [jnp.full((16,), 2, jnp.int32),
                                       jnp.full((16,), e, jnp.int32)])
            ws = plsc.bitcast(wi, jnp.float32)
            for j in range(HC // 16):
                slc = pl.ds(j * 16, 16)
                rows[e, slc] = rows[e, slc] * ws

        pltpu.sync_copy(rows, acc_sp.at[pk.at[1]], add=True)

    for b in range(NBUF):
        idx_start(b, pks[b], isems[b])
    for b in range(GLA):
        idx_wait(pks[b], isems[b])
        gstart(pks[b], rowss[b], gsems[b])

    @pl.loop(0, NCHUNK // NBUF)
    def _(g):
        c0 = g * NBUF
        for b in range(NBUF):
            c = c0 + b
            gwait(pks[b], rowss[b], gsems[b])
            process(pks[b], rowss[b])

            @pl.when(c + NBUF < NCHUNK)
            def _():
                idx_start(c + NBUF, pks[b], isems[b])

            b3 = (b + GLA) % NBUF

            @pl.when(c + GLA < NCHUNK)
            def _():
                idx_wait(pks[b3], isems[b3])
                gstart(pks[b3], rowss[b3], gsems[b3])

    plsc.subcore_barrier()

    @pl.loop(0, RPS // ZR)
    def _(t):
        r0 = sid * RPS + t * ZR
        pltpu.sync_copy(acc_sp.at[pl.ds(r0, ZR)],
                        out_hbm.at[cid].at[pl.ds(r0, ZR)])


def _sc_compiler_params():
    cp = pltpu.CompilerParams()
    if "needs_layout_passes" in pltpu.CompilerParams.__dataclass_fields__:
        cp = dataclasses.replace(cp, needs_layout_passes=False)
    if "use_tc_tiling_on_sc" in pltpu.CompilerParams.__dataclass_fields__:
        cp = dataclasses.replace(cp, use_tc_tiling_on_sc=False)
    return cp


def _sc_edge(m, epk):
    mesh = plsc.VectorSubcoreMesh(core_axis_name="c", subcore_axis_name="s")
    f = pl.kernel(
        _edge_body,
        out_type=jax.ShapeDtypeStruct((NC, NP, HC), jnp.float32),
        mesh=mesh,
        compiler_params=_sc_compiler_params(),
        scratch_types=(
            [pltpu.VMEM((3, K), jnp.int32) for _ in range(NBUF)]
            + [pltpu.VMEM((K, HC), jnp.float32) for _ in range(NBUF)]
            + [pltpu.VMEM((ZR, HC), jnp.float32),
               pltpu.VMEM_SHARED((NP, HC), jnp.float32)]
            + [pltpu.SemaphoreType.DMA for _ in range(2 * NBUF)]
        ),
    )
    return f(m, epk)


def _pack_edges(src, dst, w):
    return jnp.stack([
        src.reshape(E // K, K),
        dst.reshape(E // K, K),
        lax.bitcast_convert_type(w, jnp.int32).reshape(E // K, K),
    ], axis=1)


def _prep_conv_w(weight):
    return jnp.pad(weight, ((0, 0), (0, HP - H), (0, HP - H)))


def _prep_gru_w(w):
    blocks = [jnp.pad(w[g * H:(g + 1) * H, :].T,
                      ((0, HP - H), (0, HP - H))) for g in range(3)]
    return jnp.concatenate(blocks, axis=1)


def _prep_gru_b(b):
    bs = [jnp.pad(b[g * H:(g + 1) * H], (0, HP - H)) for g in range(3)]
    return jnp.concatenate(bs)[None, :]


def _ggc(xp, epk, conv_w, wih, whh, bih, bhh, next_w0):
    m = _tc_matmul(xp, conv_w[0])
    for i in range(L):
        gh = _tc_gh(xp, whh, bhh)
        p = _sc_edge(m, epk)
        if i + 1 < L:
            xp, m = _tc_post(p, xp, gh, wih, bih, conv_w[i + 1])
        elif next_w0 is not None:
            xp, m = _tc_post(p, xp, gh, wih, bih, next_w0)
        else:
            xp = _tc_post_final(p, xp, gh, wih, bih)
    return xp


def kernel(x, edge_index, edge_attr, enc_weight, enc_w_ih, enc_w_hh,
           enc_b_ih, enc_b_hh, dec_weight, dec_w_ih, dec_w_hh,
           dec_b_ih, dec_b_hh):
    src = edge_index[0].astype(jnp.int32)
    dst = edge_index[1].astype(jnp.int32)
    w = edge_attr.astype(jnp.float32)
    epk = _pack_edges(src, dst, w)

    xp = jnp.pad(x, ((0, 0), (0, HP - D_IN)))

    h = _ggc(xp, epk, _prep_conv_w(enc_weight),
             _prep_gru_w(enc_w_ih), _prep_gru_w(enc_w_hh),
             _prep_gru_b(enc_b_ih), _prep_gru_b(enc_b_hh), None)
    r = _ggc(h, epk, _prep_conv_w(dec_weight),
             _prep_gru_w(dec_w_ih), _prep_gru_w(dec_w_hh),
             _prep_gru_b(dec_b_ih), _prep_gru_b(dec_b_hh), None)
    return h[:, :H], r[:, :H]
